# baseline (speedup 1.0000x reference)
import functools
import math

import jax
import jax.numpy as jnp
from jax import lax
from jax.experimental import pallas as pl
from jax.experimental.pallas import tpu as pltpu
from jax.experimental.pallas import tpu_sc as plsc

_B, _N, _K, _ET = 4, 256, 128, 1024
_TOT = _B * _N * _N
_NC, _NS = 2, 16
_NW = _NC * _NS
_CHUNK = _TOT // _NW
_LANES = 16

_ROWS = 16384


def _sc_affine_kernel(x_hbm, et_hbm, mul_hbm, bias_hbm, out_hbm,
                      xv, etv, mulv, biasv, outv):
    wid = lax.axis_index("s") * _NC + lax.axis_index("c")
    base = wid * _CHUNK
    pltpu.sync_copy(mul_hbm, mulv)
    pltpu.sync_copy(bias_hbm, biasv)
    pltpu.sync_copy(x_hbm.at[pl.ds(base, _CHUNK)], xv)
    pltpu.sync_copy(et_hbm.at[pl.ds(base, _CHUNK)], etv)

    unroll = 8

    def step(i, carry):
        for u in range(unroll):
            sl = pl.ds((i * unroll + u) * _LANES, _LANES)
            idx = etv[sl]
            m = plsc.load_gather(mulv, [idx])
            b = plsc.load_gather(biasv, [idx])
            outv[sl] = m * xv[sl] + b
        return carry

    lax.fori_loop(0, _CHUNK // (_LANES * unroll), step, 0)
    pltpu.sync_copy(outv, out_hbm.at[pl.ds(base, _CHUNK)])


def _sc_affine(x_flat, et_flat, mul_flat, bias_flat):
    mesh = plsc.VectorSubcoreMesh(core_axis_name="c", subcore_axis_name="s")
    kern = functools.partial(
        pl.kernel,
        mesh=mesh,
        compiler_params=pltpu.CompilerParams(needs_layout_passes=False),
        out_type=jax.ShapeDtypeStruct((_TOT,), jnp.float32),
        scratch_types=[
            pltpu.VMEM((_CHUNK,), jnp.float32),
            pltpu.VMEM((_CHUNK,), jnp.int32),
            pltpu.VMEM((_ET,), jnp.float32),
            pltpu.VMEM((_ET,), jnp.float32),
            pltpu.VMEM((_CHUNK,), jnp.float32),
        ],
    )(_sc_affine_kernel)
    return kern(x_flat, et_flat, mul_flat, bias_flat)


_RB = 64


def _tc_gauss_kernel(xe_ref, mean_ref, std_ref, out_ref):
    mean = mean_ref[...].reshape(1, 1, _K)
    s = jnp.abs(std_ref[...]).reshape(1, 1, _K) + 1e-5
    istd = 1.0 / s
    coef = istd * (1.0 / math.sqrt(2.0 * math.pi))
    half = istd * math.sqrt(0.5)
    xv = xe_ref[...]
    t = (xv[:, :, None] - mean) * half
    out_ref[...] = jnp.exp(-(t * t)) * coef


def _tc_gauss(xe, means, stds):
    rows = _B * _N
    return pl.pallas_call(
        _tc_gauss_kernel,
        grid=(rows // _RB,),
        in_specs=[
            pl.BlockSpec((_RB, _N), lambda i: (i, 0)),
            pl.BlockSpec((1, _K), lambda i: (0, 0)),
            pl.BlockSpec((1, _K), lambda i: (0, 0)),
        ],
        out_specs=pl.BlockSpec((_RB, _N, _K), lambda i: (i, 0, 0)),
        out_shape=jax.ShapeDtypeStruct((rows, _N, _K), jnp.float32),
    )(xe.reshape(rows, _N), means, stds)


@jax.jit
def kernel(x, edge_type, means, stds, mul, bias):
    xe = _sc_affine(
        x.reshape(_TOT),
        edge_type.reshape(_TOT),
        mul.reshape(_ET),
        bias.reshape(_ET),
    )
    out = _tc_gauss(xe, means, stds)
    return out.reshape(_B, _N, _N, _K)


# speedup vs baseline: 1.0002x; 1.0002x over previous
import functools
import math

import jax
import jax.numpy as jnp
from jax import lax
from jax.experimental import pallas as pl
from jax.experimental.pallas import tpu as pltpu
from jax.experimental.pallas import tpu_sc as plsc

_B, _N, _K, _ET = 4, 256, 128, 1024
_TOT = _B * _N * _N
_NC, _NS = 2, 16
_NW = _NC * _NS
_CHUNK = _TOT // _NW
_LANES = 16

_ROWS = 16384


def _sc_affine_kernel(x_hbm, et_hbm, mul_hbm, bias_hbm, out_hbm,
                      xv, etv, mulv, biasv, outv):
    wid = lax.axis_index("s") * _NC + lax.axis_index("c")
    base = wid * _CHUNK
    pltpu.sync_copy(mul_hbm, mulv)
    pltpu.sync_copy(bias_hbm, biasv)
    pltpu.sync_copy(x_hbm.at[pl.ds(base, _CHUNK)], xv)
    pltpu.sync_copy(et_hbm.at[pl.ds(base, _CHUNK)], etv)

    unroll = 8

    def step(i, carry):
        for u in range(unroll):
            sl = pl.ds((i * unroll + u) * _LANES, _LANES)
            idx = etv[sl]
            m = plsc.load_gather(mulv, [idx])
            b = plsc.load_gather(biasv, [idx])
            outv[sl] = m * xv[sl] + b
        return carry

    lax.fori_loop(0, _CHUNK // (_LANES * unroll), step, 0)
    pltpu.sync_copy(outv, out_hbm.at[pl.ds(base, _CHUNK)])


def _sc_affine(x_flat, et_flat, mul_flat, bias_flat):
    mesh = plsc.VectorSubcoreMesh(core_axis_name="c", subcore_axis_name="s")
    kern = functools.partial(
        pl.kernel,
        mesh=mesh,
        compiler_params=pltpu.CompilerParams(needs_layout_passes=False),
        out_type=jax.ShapeDtypeStruct((_TOT,), jnp.float32),
        scratch_types=[
            pltpu.VMEM((_CHUNK,), jnp.float32),
            pltpu.VMEM((_CHUNK,), jnp.int32),
            pltpu.VMEM((_ET,), jnp.float32),
            pltpu.VMEM((_ET,), jnp.float32),
            pltpu.VMEM((_CHUNK,), jnp.float32),
        ],
    )(_sc_affine_kernel)
    return kern(x_flat, et_flat, mul_flat, bias_flat)


_RB = 128


def _tc_gauss_kernel(xe_ref, mean_ref, std_ref, out_ref):
    mean = mean_ref[...].reshape(1, 1, _K)
    s = jnp.abs(std_ref[...]).reshape(1, 1, _K) + 1e-5
    istd = 1.0 / s
    coef = istd * (1.0 / math.sqrt(2.0 * math.pi))
    half = istd * math.sqrt(0.5)
    xv = xe_ref[...]
    t = (xv[:, :, None] - mean) * half
    out_ref[...] = jnp.exp(-(t * t)) * coef


def _tc_gauss(xe, means, stds):
    rows = _B * _N
    return pl.pallas_call(
        _tc_gauss_kernel,
        grid=(rows // _RB,),
        in_specs=[
            pl.BlockSpec((_RB, _N), lambda i: (i, 0)),
            pl.BlockSpec((1, _K), lambda i: (0, 0)),
            pl.BlockSpec((1, _K), lambda i: (0, 0)),
        ],
        out_specs=pl.BlockSpec((_RB, _N, _K), lambda i: (i, 0, 0)),
        out_shape=jax.ShapeDtypeStruct((rows, _N, _K), jnp.float32),
    )(xe.reshape(rows, _N), means, stds)


@jax.jit
def kernel(x, edge_type, means, stds, mul, bias):
    xe = _sc_affine(
        x.reshape(_TOT),
        edge_type.reshape(_TOT),
        mul.reshape(_ET),
        bias.reshape(_ET),
    )
    out = _tc_gauss(xe, means, stds)
    return out.reshape(_B, _N, _N, _K)


# speedup vs baseline: 1.0779x; 1.0777x over previous
import functools
import math

import jax
import jax.numpy as jnp
from jax import lax
from jax.experimental import pallas as pl
from jax.experimental.pallas import tpu as pltpu
from jax.experimental.pallas import tpu_sc as plsc

_B, _N, _K, _ET = 4, 256, 128, 1024
_TOT = _B * _N * _N
_NC, _NS = 2, 16
_NW = _NC * _NS
_CHUNK = _TOT // _NW
_LANES = 16

_ROWS = 16384


_RPW = (_B * _N) // _NW


def _sc_affine_kernel(x_hbm, et_hbm, mul_hbm, bias_hbm, out_hbm,
                      xv, etv, mulv, biasv, outv):
    wid = lax.axis_index("s") * _NC + lax.axis_index("c")
    base = wid * _RPW
    pltpu.sync_copy(mul_hbm, mulv)
    pltpu.sync_copy(bias_hbm, biasv)
    pltpu.sync_copy(x_hbm.at[pl.ds(base, _RPW)], xv)
    pltpu.sync_copy(et_hbm.at[pl.ds(base, _RPW)], etv)

    nsl = _N // _LANES

    def step(i, carry):
        r = i // nsl
        sl = pl.ds((i % nsl) * _LANES, _LANES)
        idx = etv[r, sl]
        m = plsc.load_gather(mulv, [idx])
        b = plsc.load_gather(biasv, [idx])
        outv[r, sl] = m * xv[r, sl] + b
        return carry

    lax.fori_loop(0, _RPW * nsl, step, 0)
    pltpu.sync_copy(outv, out_hbm.at[pl.ds(base, _RPW)])


def _sc_affine(x2, et2, mul_flat, bias_flat):
    mesh = plsc.VectorSubcoreMesh(core_axis_name="c", subcore_axis_name="s")
    kern = functools.partial(
        pl.kernel,
        mesh=mesh,
        compiler_params=pltpu.CompilerParams(needs_layout_passes=False),
        out_type=jax.ShapeDtypeStruct((_B * _N, _N), jnp.float32),
        scratch_types=[
            pltpu.VMEM((_RPW, _N), jnp.float32),
            pltpu.VMEM((_RPW, _N), jnp.int32),
            pltpu.VMEM((_ET,), jnp.float32),
            pltpu.VMEM((_ET,), jnp.float32),
            pltpu.VMEM((_RPW, _N), jnp.float32),
        ],
    )(_sc_affine_kernel)
    return kern(x2, et2, mul_flat, bias_flat)


_RB = 128


def _tc_gauss_kernel(xe_ref, mean_ref, std_ref, out_ref):
    mean = mean_ref[...].reshape(1, 1, _K)
    s = jnp.abs(std_ref[...]).reshape(1, 1, _K) + 1e-5
    istd = 1.0 / s
    coef = istd * (1.0 / math.sqrt(2.0 * math.pi))
    half = istd * math.sqrt(0.5)
    xv = xe_ref[...]
    t = (xv[:, :, None] - mean) * half
    out_ref[...] = jnp.exp(-(t * t)) * coef


def _tc_gauss(xe, means, stds):
    rows = _B * _N
    return pl.pallas_call(
        _tc_gauss_kernel,
        grid=(rows // _RB,),
        in_specs=[
            pl.BlockSpec((_RB, _N), lambda i: (i, 0)),
            pl.BlockSpec((1, _K), lambda i: (0, 0)),
            pl.BlockSpec((1, _K), lambda i: (0, 0)),
        ],
        out_specs=pl.BlockSpec((_RB, _N, _K), lambda i: (i, 0, 0)),
        out_shape=jax.ShapeDtypeStruct((rows, _N, _K), jnp.float32),
    )(xe.reshape(rows, _N), means, stds)


@jax.jit
def kernel(x, edge_type, means, stds, mul, bias):
    xe = _sc_affine(
        x.reshape(_B * _N, _N),
        edge_type.reshape(_B * _N, _N),
        mul.reshape(_ET),
        bias.reshape(_ET),
    )
    out = _tc_gauss(xe, means, stds)
    return out.reshape(_B, _N, _N, _K)


# speedup vs baseline: 1.1106x; 1.0304x over previous
import functools
import math

import jax
import jax.numpy as jnp
from jax import lax
from jax.experimental import pallas as pl
from jax.experimental.pallas import tpu as pltpu
from jax.experimental.pallas import tpu_sc as plsc

_B, _N, _K, _ET = 4, 256, 128, 1024
_TOT = _B * _N * _N
_NC, _NS = 2, 16
_NW = _NC * _NS
_CHUNK = _TOT // _NW
_LANES = 16

_ROWS = 16384


_RPW = (_B * _N) // _NW


def _sc_affine_kernel(x_hbm, et_hbm, mul_hbm, bias_hbm, out_hbm,
                      xv, etv, mulv, biasv, outv):
    wid = lax.axis_index("s") * _NC + lax.axis_index("c")
    base = wid * _RPW
    pltpu.sync_copy(mul_hbm, mulv)
    pltpu.sync_copy(bias_hbm, biasv)
    pltpu.sync_copy(x_hbm.at[pl.ds(base, _RPW)], xv)
    pltpu.sync_copy(et_hbm.at[pl.ds(base, _RPW)], etv)

    nsl = _N // _LANES

    @plsc.parallel_loop(0, _RPW * nsl, unroll=8)
    def _(i):
        r = i // nsl
        sl = pl.ds((i % nsl) * _LANES, _LANES)
        idx = etv[r, sl]
        m = plsc.load_gather(mulv, [idx])
        b = plsc.load_gather(biasv, [idx])
        outv[r, sl] = m * xv[r, sl] + b
    pltpu.sync_copy(outv, out_hbm.at[pl.ds(base, _RPW)])


def _sc_affine(x2, et2, mul_flat, bias_flat):
    mesh = plsc.VectorSubcoreMesh(core_axis_name="c", subcore_axis_name="s")
    kern = functools.partial(
        pl.kernel,
        mesh=mesh,
        compiler_params=pltpu.CompilerParams(needs_layout_passes=False),
        out_type=jax.ShapeDtypeStruct((_B * _N, _N), jnp.float32),
        scratch_types=[
            pltpu.VMEM((_RPW, _N), jnp.float32),
            pltpu.VMEM((_RPW, _N), jnp.int32),
            pltpu.VMEM((_ET,), jnp.float32),
            pltpu.VMEM((_ET,), jnp.float32),
            pltpu.VMEM((_RPW, _N), jnp.float32),
        ],
    )(_sc_affine_kernel)
    return kern(x2, et2, mul_flat, bias_flat)


_RB = 128


def _tc_gauss_kernel(xe_ref, mean_ref, std_ref, out_ref):
    mean = mean_ref[...].reshape(1, 1, _K)
    s = jnp.abs(std_ref[...]).reshape(1, 1, _K) + 1e-5
    istd = 1.0 / s
    coef = istd * (1.0 / math.sqrt(2.0 * math.pi))
    half = istd * math.sqrt(0.5)
    xv = xe_ref[...]
    t = (xv[:, :, None] - mean) * half
    out_ref[...] = jnp.exp(-(t * t)) * coef


def _tc_gauss(xe, means, stds):
    rows = _B * _N
    return pl.pallas_call(
        _tc_gauss_kernel,
        grid=(rows // _RB,),
        in_specs=[
            pl.BlockSpec((_RB, _N), lambda i: (i, 0)),
            pl.BlockSpec((1, _K), lambda i: (0, 0)),
            pl.BlockSpec((1, _K), lambda i: (0, 0)),
        ],
        out_specs=pl.BlockSpec((_RB, _N, _K), lambda i: (i, 0, 0)),
        out_shape=jax.ShapeDtypeStruct((rows, _N, _K), jnp.float32),
    )(xe.reshape(rows, _N), means, stds)


@jax.jit
def kernel(x, edge_type, means, stds, mul, bias):
    xe = _sc_affine(
        x.reshape(_B * _N, _N),
        edge_type.reshape(_B * _N, _N),
        mul.reshape(_ET),
        bias.reshape(_ET),
    )
    out = _tc_gauss(xe, means, stds)
    return out.reshape(_B, _N, _N, _K)


# speedup vs baseline: 1.1111x; 1.0004x over previous
import functools
import math

import jax
import jax.numpy as jnp
from jax import lax
from jax.experimental import pallas as pl
from jax.experimental.pallas import tpu as pltpu
from jax.experimental.pallas import tpu_sc as plsc

_B, _N, _K, _ET = 4, 256, 128, 1024
_TOT = _B * _N * _N
_NC, _NS = 2, 16
_NW = _NC * _NS
_CHUNK = _TOT // _NW
_LANES = 16

_ROWS = 16384


_RPW = (_B * _N) // _NW


def _sc_affine_kernel(x_hbm, et_hbm, mul_hbm, bias_hbm, out_hbm,
                      xv, etv, mulv, biasv, outv):
    wid = lax.axis_index("s") * _NC + lax.axis_index("c")
    base = wid * _RPW
    pltpu.sync_copy(mul_hbm, mulv)
    pltpu.sync_copy(bias_hbm, biasv)
    pltpu.sync_copy(x_hbm.at[pl.ds(base, _RPW)], xv)
    pltpu.sync_copy(et_hbm.at[pl.ds(base, _RPW)], etv)

    nsl = _N // _LANES

    @plsc.parallel_loop(0, _RPW * nsl, unroll=8)
    def _(i):
        r = i // nsl
        sl = pl.ds((i % nsl) * _LANES, _LANES)
        idx = etv[r, sl]
        m = plsc.load_gather(mulv, [idx])
        b = plsc.load_gather(biasv, [idx])
        outv[r, sl] = m * xv[r, sl] + b
    pltpu.sync_copy(outv, out_hbm.at[pl.ds(base, _RPW)])


def _sc_affine(x2, et2, mul_flat, bias_flat):
    mesh = plsc.VectorSubcoreMesh(core_axis_name="c", subcore_axis_name="s")
    kern = functools.partial(
        pl.kernel,
        mesh=mesh,
        compiler_params=pltpu.CompilerParams(needs_layout_passes=False),
        out_type=jax.ShapeDtypeStruct((_B * _N, _N), jnp.float32),
        scratch_types=[
            pltpu.VMEM((_RPW, _N), jnp.float32),
            pltpu.VMEM((_RPW, _N), jnp.int32),
            pltpu.VMEM((_ET,), jnp.float32),
            pltpu.VMEM((_ET,), jnp.float32),
            pltpu.VMEM((_RPW, _N), jnp.float32),
        ],
    )(_sc_affine_kernel)
    return kern(x2, et2, mul_flat, bias_flat)


_RB = 128


def _tc_gauss_kernel(xe_ref, mean_ref, std_ref, out_ref):
    mean = mean_ref[...].reshape(1, 1, _K)
    s = jnp.abs(std_ref[...]).reshape(1, 1, _K) + 1e-5
    g = math.sqrt(0.5 * math.log2(math.e)) / s
    log2coef = -jnp.log2(s) - math.log2(math.sqrt(2.0 * math.pi))
    xv = xe_ref[...]
    t = (xv[:, :, None] - mean) * g
    out_ref[...] = jnp.exp2(log2coef - t * t)


def _tc_gauss(xe, means, stds):
    rows = _B * _N
    return pl.pallas_call(
        _tc_gauss_kernel,
        grid=(rows // _RB,),
        in_specs=[
            pl.BlockSpec((_RB, _N), lambda i: (i, 0)),
            pl.BlockSpec((1, _K), lambda i: (0, 0)),
            pl.BlockSpec((1, _K), lambda i: (0, 0)),
        ],
        out_specs=pl.BlockSpec((_RB, _N, _K), lambda i: (i, 0, 0)),
        out_shape=jax.ShapeDtypeStruct((rows, _N, _K), jnp.float32),
    )(xe.reshape(rows, _N), means, stds)


@jax.jit
def kernel(x, edge_type, means, stds, mul, bias):
    xe = _sc_affine(
        x.reshape(_B * _N, _N),
        edge_type.reshape(_B * _N, _N),
        mul.reshape(_ET),
        bias.reshape(_ET),
    )
    out = _tc_gauss(xe, means, stds)
    return out.reshape(_B, _N, _N, _K)
